# baseline (speedup 1.0000x reference)
import jax
import jax.numpy as jnp
from jax import lax
from jax.experimental import pallas as pl
from jax.experimental.pallas import tpu as pltpu

N = 4096
D = 128
BLK = 1024
K = 16
EPS = 1.0
BIG = 3.0e38


def _block_kernel(a_ref, qf_ref, out_ref, b2r_ref, b2c_ref, mrow_ref):
    i = pl.program_id(0)

    @pl.when(i == 0)
    def _init():
        qf = qf_ref[...]
        b2c = jnp.sum(qf * qf, axis=1, keepdims=True)
        b2c_ref[...] = b2c
        b2r_ref[...] = jnp.transpose(b2c)
        out_ref[...] = jnp.zeros((1, 1), jnp.float32)

    a = a_ref[...] * jnp.float32(-2.0)
    ab = lax.dot_general(a, qf_ref[...], (((1,), (1,)), ((), ())),
                         preferred_element_type=jnp.float32)
    a2 = b2c_ref[pl.ds(i * BLK, BLK), :]
    b2r = b2r_ref[...]

    eye = (lax.broadcasted_iota(jnp.int32, (BLK, BLK), 0)
           == lax.broadcasted_iota(jnp.int32, (BLK, BLK), 1))

    for bi in range(N // BLK):
        @pl.when(i == bi)
        def _screen(bi=bi):
            s, e = bi * BLK, (bi + 1) * BLK
            own = jnp.where(eye, jnp.float32(BIG),
                            ab[:, s:e] + b2r[:, s:e])
            m = jnp.min(own, axis=1, keepdims=True)
            if s > 0:
                m = jnp.minimum(
                    m, jnp.min(ab[:, :s] + b2r[:, :s], axis=1, keepdims=True))
            if e < N:
                m = jnp.minimum(
                    m, jnp.min(ab[:, e:] + b2r[:, e:], axis=1, keepdims=True))
            mrow_ref[...] = m

    m_rows = mrow_ref[...] + a2

    thr = jnp.float32(EPS * EPS)

    @pl.when(jnp.min(m_rows) < thr)
    def _full():
        d2c = jnp.maximum((ab + b2r) + a2, 0.0)
        m0 = jnp.min(d2c, axis=1, keepdims=True)
        drop = jnp.maximum(EPS - jnp.sqrt(m0), 0.0)
        c = jnp.maximum(EPS - jnp.sqrt(d2c), 0.0)
        s_all = jnp.sum(c, axis=1, keepdims=True)
        cnt = jnp.sum((d2c < thr).astype(jnp.float32), axis=1, keepdims=True)
        cnt_excl = cnt - (m0 < thr).astype(jnp.float32)
        fast = s_all - drop
        need_slow = jnp.max(cnt_excl) > jnp.float32(K)

        @pl.when(jnp.logical_not(need_slow))
        def _fast():
            out_ref[...] += jnp.sum(fast, keepdims=True).reshape(1, 1) \
                * jnp.float32(1.0 / N)

        @pl.when(need_slow)
        def _slow():
            def body(_, carry):
                vals, rem, acc = carry
                m = jnp.min(vals, axis=1, keepdims=True)
                ismin = vals == m
                cmult = jnp.sum(ismin.astype(jnp.float32), axis=1,
                                keepdims=True)
                take = jnp.minimum(cmult, rem)
                acc = acc + take * jnp.maximum(EPS - jnp.sqrt(m), 0.0)
                rem = rem - take
                vals = jnp.where(ismin, BIG, vals)
                return vals, rem, acc

            rem0 = jnp.full((BLK, 1), jnp.float32(K + 1))
            acc0 = jnp.zeros((BLK, 1), jnp.float32)
            _, _, acc = lax.fori_loop(0, K + 1, body, (d2c, rem0, acc0))
            acc = acc - drop
            out_ref[...] += jnp.sum(acc, keepdims=True).reshape(1, 1) \
                * jnp.float32(1.0 / N)


@jax.jit
def kernel(q):
    out = pl.pallas_call(
        _block_kernel,
        grid=(N // BLK,),
        in_specs=[
            pl.BlockSpec((BLK, D), lambda i: (i, 0)),
            pl.BlockSpec((N, D), lambda i: (0, 0)),
        ],
        out_specs=pl.BlockSpec((1, 1), lambda i: (0, 0)),
        out_shape=jax.ShapeDtypeStruct((1, 1), jnp.float32),
        scratch_shapes=[
            pltpu.VMEM((1, N), jnp.float32),
            pltpu.VMEM((N, 1), jnp.float32),
            pltpu.VMEM((BLK, 1), jnp.float32),
        ],
    )(q, q)
    return out[0, 0]


# speedup vs baseline: 1.2083x; 1.2083x over previous
import jax
import jax.numpy as jnp
from jax import lax
from jax.experimental import pallas as pl
from jax.experimental.pallas import tpu as pltpu

N = 4096
D = 128
BLK = 1024
K = 16
EPS = 1.0
BIG = 3.0e38


def _block_kernel(a_ref, qf_ref, out_ref, b2r_ref, b2c_ref, qbf_ref):
    i = pl.program_id(0)

    @pl.when(i == 0)
    def _init():
        qf = qf_ref[...]
        b2c = jnp.sum(qf * qf, axis=1, keepdims=True)
        b2c_ref[...] = b2c
        b2r_ref[...] = jnp.transpose(b2c)
        qbf_ref[...] = qf.astype(jnp.bfloat16)
        out_ref[...] = jnp.zeros((1, 1), jnp.float32)

    a = a_ref[...] * jnp.float32(-2.0)
    a2 = b2c_ref[pl.ds(i * BLK, BLK), :]

    C = jnp.float32(63.0 / 64.0)
    abf = a.astype(jnp.bfloat16)
    ab_bf = lax.dot_general(abf, qbf_ref[...], (((1,), (1,)), ((), ())),
                            preferred_element_type=jnp.float32)
    b2rm = b2r_ref[...] * C
    row_iota = lax.broadcasted_iota(jnp.int32, (BLK, N), 0)
    col_iota = lax.broadcasted_iota(jnp.int32, (BLK, N), 1)
    is_diag = col_iota == row_iota + i * BLK
    screened = jnp.where(is_diag, jnp.float32(BIG), ab_bf + b2rm)
    m_rows = jnp.min(screened, axis=1, keepdims=True) + a2 * C

    thr = jnp.float32(EPS * EPS)

    @pl.when(jnp.min(m_rows) < thr)
    def _full():
        b2r = b2r_ref[...]
        ab = lax.dot_general(a, qf_ref[...], (((1,), (1,)), ((), ())),
                             preferred_element_type=jnp.float32)
        d2c = jnp.maximum((ab + b2r) + a2, 0.0)
        m0 = jnp.min(d2c, axis=1, keepdims=True)
        drop = jnp.maximum(EPS - jnp.sqrt(m0), 0.0)
        c = jnp.maximum(EPS - jnp.sqrt(d2c), 0.0)
        s_all = jnp.sum(c, axis=1, keepdims=True)
        cnt = jnp.sum((d2c < thr).astype(jnp.float32), axis=1, keepdims=True)
        cnt_excl = cnt - (m0 < thr).astype(jnp.float32)
        fast = s_all - drop
        need_slow = jnp.max(cnt_excl) > jnp.float32(K)

        @pl.when(jnp.logical_not(need_slow))
        def _fast():
            out_ref[...] += jnp.sum(fast, keepdims=True).reshape(1, 1) \
                * jnp.float32(1.0 / N)

        @pl.when(need_slow)
        def _slow():
            def body(_, carry):
                vals, rem, acc = carry
                m = jnp.min(vals, axis=1, keepdims=True)
                ismin = vals == m
                cmult = jnp.sum(ismin.astype(jnp.float32), axis=1,
                                keepdims=True)
                take = jnp.minimum(cmult, rem)
                acc = acc + take * jnp.maximum(EPS - jnp.sqrt(m), 0.0)
                rem = rem - take
                vals = jnp.where(ismin, BIG, vals)
                return vals, rem, acc

            rem0 = jnp.full((BLK, 1), jnp.float32(K + 1))
            acc0 = jnp.zeros((BLK, 1), jnp.float32)
            _, _, acc = lax.fori_loop(0, K + 1, body, (d2c, rem0, acc0))
            acc = acc - drop
            out_ref[...] += jnp.sum(acc, keepdims=True).reshape(1, 1) \
                * jnp.float32(1.0 / N)


@jax.jit
def kernel(q):
    out = pl.pallas_call(
        _block_kernel,
        grid=(N // BLK,),
        in_specs=[
            pl.BlockSpec((BLK, D), lambda i: (i, 0)),
            pl.BlockSpec((N, D), lambda i: (0, 0)),
        ],
        out_specs=pl.BlockSpec((1, 1), lambda i: (0, 0)),
        out_shape=jax.ShapeDtypeStruct((1, 1), jnp.float32),
        scratch_shapes=[
            pltpu.VMEM((1, N), jnp.float32),
            pltpu.VMEM((N, 1), jnp.float32),
            pltpu.VMEM((N, D), jnp.bfloat16),
        ],
    )(q, q)
    return out[0, 0]


# speedup vs baseline: 1.2331x; 1.0205x over previous
import jax
import jax.numpy as jnp
from jax import lax
from jax.experimental import pallas as pl
from jax.experimental.pallas import tpu as pltpu

N = 4096
D = 128
BLK = 1024
K = 16
EPS = 1.0
BIG = 3.0e38


def _block_kernel(a_ref, qf_ref, out_ref, b2r_ref, b2c_ref):
    i = pl.program_id(0)

    @pl.when(i == 0)
    def _init():
        qf = qf_ref[...]
        b2c = jnp.sum(qf * qf, axis=1, keepdims=True)
        b2c_ref[...] = b2c
        b2r_ref[...] = jnp.transpose(b2c)
        out_ref[...] = jnp.zeros((1, 1), jnp.float32)

    a = a_ref[...] * jnp.float32(-2.0)
    a2 = b2c_ref[pl.ds(i * BLK, BLK), :]

    ab = lax.dot_general(a, qf_ref[...], (((1,), (1,)), ((), ())),
                         preferred_element_type=jnp.float32)
    b2r = b2r_ref[...]

    thr = jnp.float32(EPS * EPS)
    below = (ab + b2r) < (thr - a2)
    cnt_blw = jnp.sum(below.astype(jnp.float32), axis=1, keepdims=True)
    big_norm = a2 * jnp.float32(2.0 ** -10) >= thr
    fire = jnp.logical_or(cnt_blw >= 2.0,
                          jnp.logical_and(cnt_blw >= 1.0, big_norm))

    @pl.when(jnp.max(fire.astype(jnp.float32)) > 0.0)
    def _full():
        d2c = jnp.maximum((ab + b2r) + a2, 0.0)
        m0 = jnp.min(d2c, axis=1, keepdims=True)
        drop = jnp.maximum(EPS - jnp.sqrt(m0), 0.0)
        c = jnp.maximum(EPS - jnp.sqrt(d2c), 0.0)
        s_all = jnp.sum(c, axis=1, keepdims=True)
        cnt = jnp.sum((d2c < thr).astype(jnp.float32), axis=1, keepdims=True)
        cnt_excl = cnt - (m0 < thr).astype(jnp.float32)
        fast = s_all - drop
        need_slow = jnp.max(cnt_excl) > jnp.float32(K)

        @pl.when(jnp.logical_not(need_slow))
        def _fast():
            out_ref[...] += jnp.sum(fast, keepdims=True).reshape(1, 1) \
                * jnp.float32(1.0 / N)

        @pl.when(need_slow)
        def _slow():
            def body(_, carry):
                vals, rem, acc = carry
                m = jnp.min(vals, axis=1, keepdims=True)
                ismin = vals == m
                cmult = jnp.sum(ismin.astype(jnp.float32), axis=1,
                                keepdims=True)
                take = jnp.minimum(cmult, rem)
                acc = acc + take * jnp.maximum(EPS - jnp.sqrt(m), 0.0)
                rem = rem - take
                vals = jnp.where(ismin, BIG, vals)
                return vals, rem, acc

            rem0 = jnp.full((BLK, 1), jnp.float32(K + 1))
            acc0 = jnp.zeros((BLK, 1), jnp.float32)
            _, _, acc = lax.fori_loop(0, K + 1, body, (d2c, rem0, acc0))
            acc = acc - drop
            out_ref[...] += jnp.sum(acc, keepdims=True).reshape(1, 1) \
                * jnp.float32(1.0 / N)


@jax.jit
def kernel(q):
    out = pl.pallas_call(
        _block_kernel,
        grid=(N // BLK,),
        in_specs=[
            pl.BlockSpec((BLK, D), lambda i: (i, 0)),
            pl.BlockSpec((N, D), lambda i: (0, 0)),
        ],
        out_specs=pl.BlockSpec((1, 1), lambda i: (0, 0)),
        out_shape=jax.ShapeDtypeStruct((1, 1), jnp.float32),
        scratch_shapes=[
            pltpu.VMEM((1, N), jnp.float32),
            pltpu.VMEM((N, 1), jnp.float32),
        ],
    )(q, q)
    return out[0, 0]
